# baseline (speedup 1.0000x reference)
import functools

import jax
import jax.numpy as jnp
from jax import lax
from jax.experimental import pallas as pl
from jax.experimental.pallas import tpu as pltpu
from jax.experimental.pallas import tpu_sc as plsc

_NC = 2
_NS = 16
_NW = _NC * _NS


@functools.lru_cache(maxsize=None)
def _make_lookup(nh, nb, d, bc):
    n_chunks = nb // bc
    tasks = nh * n_chunks
    per_w = tasks // _NW
    assert tasks % _NW == 0 and d % 16 == 0
    mesh = plsc.VectorSubcoreMesh(core_axis_name="c", subcore_axis_name="s")
    ngrp = bc // 16

    @functools.partial(
        pl.kernel,
        mesh=mesh,
        out_type=jax.ShapeDtypeStruct((nh, d, nb), jnp.float32),
        scratch_types=[
            pltpu.VMEM((bc,), jnp.int32),
            pltpu.VMEM((bc,), jnp.int32),
            pltpu.VMEM((bc,), jnp.int32),
            pltpu.VMEM((bc, 2 * d), jnp.float32),
            pltpu.VMEM((d, bc), jnp.float32),
            pltpu.SemaphoreType.DMA,
            pltpu.SemaphoreType.DMA,
        ],
        compiler_params=pltpu.CompilerParams(
            use_tc_tiling_on_sc=True, needs_layout_passes=False
        ),
    )
    def lookup(tab, idxt, out, idx_v, gid_v, par_v, rows_v, outt_v, gsem, wsem):
        wid = lax.axis_index("s") * _NC + lax.axis_index("c")
        lane = lax.iota(jnp.int32, 16)

        def task_body(t, carry):
            task = wid * per_w + t
            h = task // n_chunks
            b0 = (task % n_chunks) * bc
            pltpu.sync_copy(idxt.at[h, pl.ds(b0, bc)], idx_v)
            for g in range(ngrp):
                iv = idx_v[pl.ds(g * 16, 16)]
                gid_v[pl.ds(g * 16, 16)] = iv >> 1
                par_v[pl.ds(g * 16, 16)] = (iv & 1) * d
            pltpu.async_copy(tab.at[gid_v], rows_v, gsem).wait()
            for g in range(ngrp):
                rows16 = lane + g * 16
                cols16 = par_v[pl.ds(g * 16, 16)]

                def col_body(c, cols):
                    vals = plsc.load_gather(rows_v, [rows16, cols])
                    plsc.store_scatter(
                        outt_v, [jnp.full((16,), c, jnp.int32), rows16], vals
                    )
                    return cols + 1

                lax.fori_loop(0, d, col_body, cols16)
            pltpu.async_copy(outt_v, out.at[h, :, pl.ds(b0, bc)], wsem).wait()
            return carry

        lax.fori_loop(0, per_w, task_body, 0)

    return lookup


def kernel(input_, weight):
    b, h = input_.shape
    v, d = weight.shape
    tab = weight.reshape(v // 2, 2 * d)
    idxt = input_.T
    outt = _make_lookup(h, b, d, 256)(tab, idxt)
    return jnp.transpose(outt, (2, 0, 1))
